# baseline (speedup 1.0000x reference)
import functools

import jax
import jax.numpy as jnp
from jax import lax
from jax.experimental import pallas as pl
from jax.experimental.pallas import tpu as pltpu
from jax.experimental.pallas import tpu_sc as plsc

V = 1000
VP = 1024
D = 128
B = 4096

_NC = 2
_NS = 16
_NW = _NC * _NS
_BPW = B // _NW

_CH = 8
_NCHUNK = _BPW // _CH


@functools.cache
def _sc_kernels():
    mesh = plsc.VectorSubcoreMesh(core_axis_name="c", subcore_axis_name="s")

    @functools.partial(
        pl.kernel,
        mesh=mesh,
        out_type=jax.ShapeDtypeStruct((B, D), jnp.float32),
        scratch_types=[
            pltpu.VMEM((_BPW,), jnp.int32),
            pltpu.VMEM((_BPW, D), jnp.float32),
            pltpu.SemaphoreType.DMA,
        ],
    )
    def gather_ctx(table_hbm, idx_hbm, out_hbm, idx_v, rows_v, sem):
        wid = lax.axis_index("s") * _NC + lax.axis_index("c")
        base = wid * _BPW
        pltpu.sync_copy(idx_hbm.at[pl.ds(base, _BPW)], idx_v)
        pltpu.async_copy(table_hbm.at[idx_v], rows_v, sem).wait()
        pltpu.sync_copy(rows_v, out_hbm.at[pl.ds(base, _BPW)])

    @functools.partial(
        pl.kernel,
        mesh=mesh,
        out_type=jax.ShapeDtypeStruct((B, B), jnp.float32),
        scratch_types=[
            pltpu.VMEM((_BPW,), jnp.int32),
            pltpu.VMEM((2, _CH, B), jnp.float32),
            pltpu.SemaphoreType.DMA,
            pltpu.SemaphoreType.DMA,
        ],
    )
    def gather_rows(tc_hbm, idx_hbm, out_hbm, idx_v, rows_v, sem0, sem1):
        wid = lax.axis_index("s") * _NC + lax.axis_index("c")
        base = wid * _BPW
        sems = (sem0, sem1)
        pltpu.sync_copy(idx_hbm.at[pl.ds(base, _BPW)], idx_v)
        cp = pltpu.async_copy(
            tc_hbm.at[idx_v.at[pl.ds(0, _CH)]], rows_v.at[0], sems[0])
        for c in range(_NCHUNK):
            b = c & 1
            nxt = None
            if c + 1 < _NCHUNK:
                nxt = pltpu.async_copy(
                    tc_hbm.at[idx_v.at[pl.ds((c + 1) * _CH, _CH)]],
                    rows_v.at[1 - b], sems[1 - b])
            cp.wait()
            pltpu.sync_copy(
                rows_v.at[b], out_hbm.at[pl.ds(base + c * _CH, _CH)])
            cp = nxt

    return gather_ctx, gather_rows


_CB = 1024


def _score_body(e_ref, ectx_ref, out_ref):
    x = lax.dot_general(
        e_ref[...], ectx_ref[...],
        (((1,), (1,)), ((), ())),
        preferred_element_type=jnp.float32,
    )
    out_ref[...] = jnp.minimum(x, 0.0) - jnp.log1p(jnp.exp(-jnp.abs(x)))


def _scores(ep, ectx):
    return pl.pallas_call(
        _score_body,
        grid=(B // _CB,),
        in_specs=[
            pl.BlockSpec((VP, D), lambda j: (0, 0)),
            pl.BlockSpec((_CB, D), lambda j: (j, 0)),
        ],
        out_specs=pl.BlockSpec((VP, _CB), lambda j: (0, j)),
        out_shape=jax.ShapeDtypeStruct((VP, B), jnp.float32),
    )(ep, ectx)


def kernel(center_id, context_id, emb_table):
    gather_ctx, gather_rows = _sc_kernels()
    ep = jnp.zeros((VP, D), jnp.float32).at[:V].set(emb_table)
    ectx = gather_ctx(ep, context_id)
    tc = _scores(ep, ectx)
    return gather_rows(tc, center_id)
